# baseline (speedup 1.0000x reference)
import functools
import math

import jax
import jax.numpy as jnp
from jax import lax
from jax.experimental import pallas as pl
from jax.experimental.pallas import tpu as pltpu
from jax.experimental.pallas import tpu_sc as plsc

N = 10000
E = 320000
D = 128
NPAD = 10112
TRASH = N
NW = 32
EPW = E // NW
CH = 96
PEND = 53 * 2 * CH
LCH = 800
NLCH = EPW // LCH
LTAIL = EPW - NLCH * LCH
ZROWS = NPAD // 16

@functools.cache
def _get_sc_conv():
    mesh = plsc.VectorSubcoreMesh(core_axis_name="c", subcore_axis_name="s",
                                  num_cores=2, num_subcores=16)
    return functools.partial(
        pl.kernel,
        out_type=jax.ShapeDtypeStruct((2 * NPAD, D), jnp.float32),
        mesh=mesh,
        compiler_params=pltpu.CompilerParams(needs_layout_passes=False),
        scratch_types=[
        pltpu.VMEM_SHARED((NPAD, D), jnp.float32),
        pltpu.VMEM((N,), jnp.int32),
        pltpu.VMEM((PEND,), jnp.int32),
        pltpu.VMEM((CH,), jnp.int32),
        pltpu.VMEM((CH,), jnp.int32),
        pltpu.VMEM((CH, D), jnp.float32),
        pltpu.VMEM((CH,), jnp.int32),
        pltpu.VMEM((CH,), jnp.int32),
        pltpu.VMEM((CH, D), jnp.float32),
        pltpu.VMEM((LCH,), jnp.int32),
        pltpu.VMEM((LCH,), jnp.int32),
        pltpu.VMEM((LCH,), jnp.int32),
        pltpu.VMEM((LCH,), jnp.int32),
        pltpu.SemaphoreType.DMA,
        pltpu.SemaphoreType.DMA,
        pltpu.SemaphoreType.DMA,
        pltpu.SemaphoreType.DMA,
        ],
    )(_sc_conv_body)


def _sc_conv_body(x_hbm, src_hbm, dst_hbm, sel_hbm, z_hbm, out_hbm,
                  acc, sel_v, pend, sva, dva, rva, svb, dvb, rvb,
                  ias, iad, ibs, ibd, sema, semb, sia, sib):
    c = lax.axis_index("c")
    s = lax.axis_index("s")
    wid = c * 16 + s

    with jax.named_scope("sc_zero"):
        pltpu.sync_copy(z_hbm, acc.at[pl.ds(s * ZROWS, ZROWS)])
        pltpu.sync_copy(sel_hbm, sel_v)
        plsc.subcore_barrier()

    def compact_groups(n_groups, cnt, sv, dv):
        for j in range(n_groups):
            s16 = sv[pl.ds(j * 16, 16)]
            d16 = dv[pl.ds(j * 16, 16)]
            ss = plsc.load_gather(sel_v, [s16])
            sd = plsc.load_gather(sel_v, [d16])
            live = (ss > 0) & (sd > 0)
            packed = jnp.left_shift(s16, 16) | d16
            plsc.store_compressed(pend.at[pl.ds(cnt, 16)], packed, mask=live)
            cnt = cnt + jnp.sum(live.astype(jnp.int32))
        return cnt

    def start_iload(t, is_, id_, sem):
        base = wid * EPW + t * LCH
        pltpu.async_copy(src_hbm.at[pl.ds(base, LCH)], is_, sem)
        pltpu.async_copy(dst_hbm.at[pl.ds(base, LCH)], id_, sem)

    def wait_iload(is_, id_, sem):
        pltpu.make_async_copy(src_hbm.at[pl.ds(0, LCH)], is_, sem).wait()
        pltpu.make_async_copy(src_hbm.at[pl.ds(0, LCH)], id_, sem).wait()

    start_iload(0, ias, iad, sia)

    def pa_pair(g, cnt):
        start_iload(2 * g + 1, ibs, ibd, sib)
        wait_iload(ias, iad, sia)
        cnt = compact_groups(LCH // 16, cnt, ias, iad)

        @pl.when(2 * g + 2 < NLCH)
        def _():
            start_iload(2 * g + 2, ias, iad, sia)

        wait_iload(ibs, ibd, sib)
        cnt = compact_groups(LCH // 16, cnt, ibs, ibd)
        return cnt

    with jax.named_scope("sc_phaseA"):
        cnt = lax.fori_loop(0, NLCH // 2, pa_pair, jnp.int32(0))
        base = wid * EPW + NLCH * LCH
        pltpu.sync_copy(src_hbm.at[pl.ds(base, LTAIL)], ias.at[pl.ds(0, LTAIL)])
        pltpu.sync_copy(dst_hbm.at[pl.ds(base, LTAIL)], iad.at[pl.ds(0, LTAIL)])
        cnt = compact_groups(LTAIL // 16, cnt, ias, iad)

    npair = (cnt + 2 * CH - 1) // (2 * CH)
    ngrp = npair * (2 * CH // 16)

    def pad_body(g, carry):
        o = g * 16
        v = pend[pl.ds(o, 16)]
        idx = o + lax.iota(jnp.int32, 16)
        pend[pl.ds(o, 16)] = jnp.where(idx >= cnt, TRASH + (idx & 63), v)
        return carry

    lax.fori_loop(cnt // 16, ngrp, pad_body, 0)

    def unpack(ch, sv, dv):
        for j in range(CH // 16):
            v = pend[pl.ds(ch * CH + j * 16, 16)]
            sv[pl.ds(j * 16, 16)] = jnp.right_shift(v, 16)
            dv[pl.ds(j * 16, 16)] = v & 0xFFFF

    @pl.when(npair > 0)
    def _():
        unpack(0, sva, dva)
        pltpu.async_copy(x_hbm.at[sva], rva, sema)

    def pair_body(g, carry):
        unpack(2 * g + 1, svb, dvb)
        desc_b = pltpu.async_copy(x_hbm.at[svb], rvb, semb)
        pltpu.make_async_copy(x_hbm.at[sva], rva, sema).wait()
        pltpu.sync_copy(rva, acc.at[dva], add=True)

        @pl.when(2 * g + 2 < 2 * npair)
        def _():
            unpack(2 * g + 2, sva, dva)
            pltpu.async_copy(x_hbm.at[sva], rva, sema)

        desc_b.wait()
        pltpu.sync_copy(rvb, acc.at[dvb], add=True)
        return carry

    with jax.named_scope("sc_phaseB"):
        lax.fori_loop(0, npair, pair_body, 0)
        plsc.subcore_barrier()

    with jax.named_scope("sc_dump"):
        pltpu.sync_copy(acc.at[pl.ds(s * ZROWS, ZROWS)],
                        out_hbm.at[pl.ds(c * NPAD + s * ZROWS, ZROWS)])


_NEG = -3.4e38


def _tc_layer_body(aggs_ref, x_ref, m_ref, wrT_ref, br_ref, woT_ref, g_ref, b_ref,
                   pv_ref, racc_ref, xout_ref, sel_ref, rout_ref, *, kcur, knext):
    agg = aggs_ref[0:N, :] + aggs_ref[NPAD:NPAD + N, :]
    x = x_ref[...]
    h = (jnp.dot(agg, wrT_ref[...], preferred_element_type=jnp.float32)
         + br_ref[...]
         + jnp.dot(x, woT_ref[...], preferred_element_type=jnp.float32))
    h = jnp.maximum(h, 0.0)
    m = m_ref[...]
    mean = jnp.sum(h * m, axis=0, keepdims=True) / kcur
    var = jnp.sum(((h - mean) ** 2) * m, axis=0, keepdims=True) / kcur
    hb = (h - mean) * lax.rsqrt(var + 1e-5) * g_ref[...] + b_ref[...]

    pv = pv_ref[...]
    pnorm = jnp.sqrt(jnp.sum(pv * pv)) + 1e-16
    score = jnp.dot(hb, pv, preferred_element_type=jnp.float32) / pnorm
    score = jnp.where(m > 0, score, _NEG)

    bits = lax.bitcast_convert_type(score, jnp.int32)
    skey = jnp.where(bits < 0, bits ^ jnp.int32(0x7FFFFFFF), bits)

    def t_body(i, t):
        cand = t + jnp.left_shift(jnp.int32(1), 30 - i)
        cnt = jnp.sum((skey >= cand).astype(jnp.int32))
        return jnp.where(cnt >= knext, cand, t)

    cnt_pos = jnp.sum((skey >= 0).astype(jnp.int32))
    t0 = jnp.where(cnt_pos >= knext, jnp.int32(0), jnp.int32(-2**31))
    t = lax.fori_loop(0, 31, t_body, t0)

    gt = skey > t
    tie = skey == t
    need = knext - jnp.sum(gt.astype(jnp.int32))
    iot = lax.broadcasted_iota(jnp.int32, (N, 1), 0)

    def i_body(i, mth):
        cand = mth + jnp.left_shift(jnp.int32(1), 13 - i)
        cnt = jnp.sum((tie & (iot <= cand)).astype(jnp.int32))
        return jnp.where(cnt < need, cand, mth)

    mth = lax.fori_loop(0, 14, i_body, jnp.int32(-1))
    sel = gt | (tie & (iot <= mth + 1))

    xn = hb * jnp.tanh(score) * sel.astype(jnp.float32)
    xout_ref[...] = xn
    sel_ref[...] = sel.astype(jnp.int32)
    gmax = jnp.max(jnp.where(sel, xn, _NEG), axis=0)
    gmean = jnp.sum(xn, axis=0) / knext
    rout_ref[...] = racc_ref[...] + jnp.concatenate([gmax, gmean]).reshape(1, 256)


def _make_tc_layer(kcur, knext):
    return pl.pallas_call(
        functools.partial(_tc_layer_body, kcur=kcur, knext=knext),
        out_shape=(
            jax.ShapeDtypeStruct((N, D), jnp.float32),
            jax.ShapeDtypeStruct((N, 1), jnp.int32),
            jax.ShapeDtypeStruct((1, 256), jnp.float32),
        ),
    )


def _tc_head_body(r_ref, w1T_ref, b1_ref, w2T_ref, b2_ref, w3T_ref, b3_ref, out_ref):
    h = jnp.maximum(jnp.dot(r_ref[...], w1T_ref[...],
                            preferred_element_type=jnp.float32) + b1_ref[...], 0.0)
    h = jnp.maximum(jnp.dot(h, w2T_ref[...],
                            preferred_element_type=jnp.float32) + b2_ref[...], 0.0)
    l = jnp.dot(h, w3T_ref[...], preferred_element_type=jnp.float32) + b3_ref[...]
    mx = jnp.max(l, axis=1, keepdims=True)
    lse = jnp.log(jnp.sum(jnp.exp(l - mx), axis=1, keepdims=True)) + mx
    out_ref[...] = l - lse


_tc_head = pl.pallas_call(
    _tc_head_body,
    out_shape=jax.ShapeDtypeStruct((1, 10), jnp.float32),
)


def kernel(x, edge_index, batch, Wrel, brel, Wroot, gamma, beta, pvec, W1, b1, W2, b2, W3, b3):
    src = edge_index[0].astype(jnp.int32)
    dst = edge_index[1].astype(jnp.int32)
    sel = jnp.ones((N,), jnp.int32)
    mask = jnp.ones((N, 1), jnp.float32)
    racc = jnp.zeros((1, 256), jnp.float32)
    zrows = jnp.zeros((ZROWS, D), jnp.float32)
    xcur = x
    kcur = N
    for i in range(5):
        knext = int(math.ceil(kcur * 0.5))
        aggs = _get_sc_conv()(xcur, src, dst, sel, zrows)
        xcur, seli, racc = _make_tc_layer(kcur, knext)(
            aggs, xcur, mask,
            Wrel[i].T, brel[i].reshape(1, D), Wroot[i].T,
            gamma[i].reshape(1, D), beta[i].reshape(1, D),
            pvec[i].reshape(D, 1), racc)
        sel = seli.reshape(N)
        mask = seli.astype(jnp.float32)
        kcur = knext
    return _tc_head(racc, W1.T, b1.reshape(1, 128), W2.T, b2.reshape(1, 64),
                    W3.T, b3.reshape(1, 10))


# speedup vs baseline: 2.0189x; 2.0189x over previous
import functools
import math

import jax
import jax.numpy as jnp
from jax import lax
from jax.experimental import pallas as pl
from jax.experimental.pallas import tpu as pltpu
from jax.experimental.pallas import tpu_sc as plsc

N = 10000
E = 320000
D = 128
NPAD = 10112
TRASH = N
NW = 32
EPW = E // NW
CH = 96
PEND = 53 * 2 * CH
LCH = 800
NLCH = EPW // LCH
LTAIL = EPW - NLCH * LCH
ZROWS = NPAD // 16

@functools.cache
def _get_sc_conv():
    mesh = plsc.VectorSubcoreMesh(core_axis_name="c", subcore_axis_name="s",
                                  num_cores=2, num_subcores=16)
    return functools.partial(
        pl.kernel,
        out_type=jax.ShapeDtypeStruct((2 * NPAD, D), jnp.float32),
        mesh=mesh,
        compiler_params=pltpu.CompilerParams(needs_layout_passes=False),
        scratch_types=[
        pltpu.VMEM_SHARED((NPAD, D), jnp.float32),
        pltpu.VMEM((N,), jnp.int32),
        pltpu.VMEM((PEND,), jnp.int32),
        pltpu.VMEM((CH,), jnp.int32),
        pltpu.VMEM((CH,), jnp.int32),
        pltpu.VMEM((CH, D), jnp.float32),
        pltpu.VMEM((CH,), jnp.int32),
        pltpu.VMEM((CH,), jnp.int32),
        pltpu.VMEM((CH, D), jnp.float32),
        pltpu.VMEM((LCH,), jnp.int32),
        pltpu.VMEM((LCH,), jnp.int32),
        pltpu.VMEM((LCH,), jnp.int32),
        pltpu.VMEM((LCH,), jnp.int32),
        pltpu.SemaphoreType.DMA,
        pltpu.SemaphoreType.DMA,
        pltpu.SemaphoreType.DMA,
        pltpu.SemaphoreType.DMA,
        ],
    )(_sc_conv_body)


def _sc_conv_body(x_hbm, src_hbm, dst_hbm, sel_hbm, z_hbm, out_hbm,
                  acc, sel_v, pend, sva, dva, rva, svb, dvb, rvb,
                  ias, iad, ibs, ibd, sema, semb, sia, sib):
    c = lax.axis_index("c")
    s = lax.axis_index("s")
    wid = c * 16 + s

    with jax.named_scope("sc_zero"):
        pltpu.sync_copy(z_hbm, acc.at[pl.ds(s * ZROWS, ZROWS)])
        pltpu.sync_copy(sel_hbm, sel_v)
        plsc.subcore_barrier()

    def compact_groups(n_groups, cnt, sv, dv):
        for j in range(n_groups):
            s16 = sv[pl.ds(j * 16, 16)]
            d16 = dv[pl.ds(j * 16, 16)]
            ss = plsc.load_gather(sel_v, [s16])
            sd = plsc.load_gather(sel_v, [d16])
            live = (ss > 0) & (sd > 0)
            packed = jnp.left_shift(s16, 16) | d16
            plsc.store_compressed(pend.at[pl.ds(cnt, 16)], packed, mask=live)
            cnt = cnt + jnp.sum(live.astype(jnp.int32))
        return cnt

    def start_iload(t, is_, id_, sem):
        base = wid * EPW + t * LCH
        pltpu.async_copy(src_hbm.at[pl.ds(base, LCH)], is_, sem)
        pltpu.async_copy(dst_hbm.at[pl.ds(base, LCH)], id_, sem)

    def wait_iload(is_, id_, sem):
        pltpu.make_async_copy(src_hbm.at[pl.ds(0, LCH)], is_, sem).wait()
        pltpu.make_async_copy(src_hbm.at[pl.ds(0, LCH)], id_, sem).wait()

    start_iload(0, ias, iad, sia)

    def pa_pair(g, cnt):
        start_iload(2 * g + 1, ibs, ibd, sib)
        wait_iload(ias, iad, sia)
        cnt = compact_groups(LCH // 16, cnt, ias, iad)

        @pl.when(2 * g + 2 < NLCH)
        def _():
            start_iload(2 * g + 2, ias, iad, sia)

        wait_iload(ibs, ibd, sib)
        cnt = compact_groups(LCH // 16, cnt, ibs, ibd)
        return cnt

    with jax.named_scope("sc_phaseA"):
        cnt = lax.fori_loop(0, NLCH // 2, pa_pair, jnp.int32(0))
        base = wid * EPW + NLCH * LCH
        pltpu.sync_copy(src_hbm.at[pl.ds(base, LTAIL)], ias.at[pl.ds(0, LTAIL)])
        pltpu.sync_copy(dst_hbm.at[pl.ds(base, LTAIL)], iad.at[pl.ds(0, LTAIL)])
        cnt = compact_groups(LTAIL // 16, cnt, ias, iad)

    npair = (cnt + 2 * CH - 1) // (2 * CH)
    ngrp = npair * (2 * CH // 16)

    def pad_body(g, carry):
        o = g * 16
        v = pend[pl.ds(o, 16)]
        idx = o + lax.iota(jnp.int32, 16)
        pad = jnp.left_shift(idx & 8191, 16) | (TRASH + (idx & 63))
        pend[pl.ds(o, 16)] = jnp.where(idx >= cnt, pad, v)
        return carry

    lax.fori_loop(cnt // 16, ngrp, pad_body, 0)

    def unpack(ch, sv, dv):
        for j in range(CH // 16):
            v = pend[pl.ds(ch * CH + j * 16, 16)]
            sv[pl.ds(j * 16, 16)] = jnp.right_shift(v, 16)
            dv[pl.ds(j * 16, 16)] = v & 0xFFFF

    @pl.when(npair > 0)
    def _():
        unpack(0, sva, dva)
        pltpu.async_copy(x_hbm.at[sva], rva, sema)

    def pair_body(g, carry):
        unpack(2 * g + 1, svb, dvb)
        desc_b = pltpu.async_copy(x_hbm.at[svb], rvb, semb)
        pltpu.make_async_copy(x_hbm.at[sva], rva, sema).wait()
        pltpu.sync_copy(rva, acc.at[dva], add=True)

        @pl.when(2 * g + 2 < 2 * npair)
        def _():
            unpack(2 * g + 2, sva, dva)
            pltpu.async_copy(x_hbm.at[sva], rva, sema)

        desc_b.wait()
        pltpu.sync_copy(rvb, acc.at[dvb], add=True)
        return carry

    with jax.named_scope("sc_phaseB"):
        lax.fori_loop(0, npair, pair_body, 0)
        plsc.subcore_barrier()

    with jax.named_scope("sc_dump"):
        pltpu.sync_copy(acc.at[pl.ds(s * ZROWS, ZROWS)],
                        out_hbm.at[pl.ds(c * NPAD + s * ZROWS, ZROWS)])


_NEG = -3.4e38


def _tc_layer_body(aggs_ref, x_ref, m_ref, wrT_ref, br_ref, woT_ref, g_ref, b_ref,
                   pv_ref, racc_ref, xout_ref, sel_ref, rout_ref, *, kcur, knext):
    agg = aggs_ref[0:N, :] + aggs_ref[NPAD:NPAD + N, :]
    x = x_ref[...]
    h = (jnp.dot(agg, wrT_ref[...], preferred_element_type=jnp.float32)
         + br_ref[...]
         + jnp.dot(x, woT_ref[...], preferred_element_type=jnp.float32))
    h = jnp.maximum(h, 0.0)
    m = m_ref[...]
    mean = jnp.sum(h * m, axis=0, keepdims=True) / kcur
    var = jnp.sum(((h - mean) ** 2) * m, axis=0, keepdims=True) / kcur
    hb = (h - mean) * lax.rsqrt(var + 1e-5) * g_ref[...] + b_ref[...]

    pv = pv_ref[...]
    pnorm = jnp.sqrt(jnp.sum(pv * pv)) + 1e-16
    score = jnp.dot(hb, pv, preferred_element_type=jnp.float32) / pnorm
    score = jnp.where(m > 0, score, _NEG)

    bits = lax.bitcast_convert_type(score, jnp.int32)
    skey = jnp.where(bits < 0, bits ^ jnp.int32(0x7FFFFFFF), bits)

    def t_body(i, t):
        cand = t + jnp.left_shift(jnp.int32(1), 30 - i)
        cnt = jnp.sum((skey >= cand).astype(jnp.int32))
        return jnp.where(cnt >= knext, cand, t)

    cnt_pos = jnp.sum((skey >= 0).astype(jnp.int32))
    t0 = jnp.where(cnt_pos >= knext, jnp.int32(0), jnp.int32(-2**31))
    t = lax.fori_loop(0, 31, t_body, t0)

    gt = skey > t
    tie = skey == t
    need = knext - jnp.sum(gt.astype(jnp.int32))
    iot = lax.broadcasted_iota(jnp.int32, (N, 1), 0)

    def i_body(i, mth):
        cand = mth + jnp.left_shift(jnp.int32(1), 13 - i)
        cnt = jnp.sum((tie & (iot <= cand)).astype(jnp.int32))
        return jnp.where(cnt < need, cand, mth)

    mth = lax.fori_loop(0, 14, i_body, jnp.int32(-1))
    sel = gt | (tie & (iot <= mth + 1))

    xn = hb * jnp.tanh(score) * sel.astype(jnp.float32)
    xout_ref[...] = xn
    sel_ref[...] = sel.astype(jnp.int32)
    gmax = jnp.max(jnp.where(sel, xn, _NEG), axis=0)
    gmean = jnp.sum(xn, axis=0) / knext
    rout_ref[...] = racc_ref[...] + jnp.concatenate([gmax, gmean]).reshape(1, 256)


def _make_tc_layer(kcur, knext):
    return pl.pallas_call(
        functools.partial(_tc_layer_body, kcur=kcur, knext=knext),
        out_shape=(
            jax.ShapeDtypeStruct((N, D), jnp.float32),
            jax.ShapeDtypeStruct((N, 1), jnp.int32),
            jax.ShapeDtypeStruct((1, 256), jnp.float32),
        ),
    )


def _tc_head_body(r_ref, w1T_ref, b1_ref, w2T_ref, b2_ref, w3T_ref, b3_ref, out_ref):
    h = jnp.maximum(jnp.dot(r_ref[...], w1T_ref[...],
                            preferred_element_type=jnp.float32) + b1_ref[...], 0.0)
    h = jnp.maximum(jnp.dot(h, w2T_ref[...],
                            preferred_element_type=jnp.float32) + b2_ref[...], 0.0)
    l = jnp.dot(h, w3T_ref[...], preferred_element_type=jnp.float32) + b3_ref[...]
    mx = jnp.max(l, axis=1, keepdims=True)
    lse = jnp.log(jnp.sum(jnp.exp(l - mx), axis=1, keepdims=True)) + mx
    out_ref[...] = l - lse


_tc_head = pl.pallas_call(
    _tc_head_body,
    out_shape=jax.ShapeDtypeStruct((1, 10), jnp.float32),
)


def kernel(x, edge_index, batch, Wrel, brel, Wroot, gamma, beta, pvec, W1, b1, W2, b2, W3, b3):
    src = edge_index[0].astype(jnp.int32)
    dst = edge_index[1].astype(jnp.int32)
    sel = jnp.ones((N,), jnp.int32)
    mask = jnp.ones((N, 1), jnp.float32)
    racc = jnp.zeros((1, 256), jnp.float32)
    zrows = jnp.zeros((ZROWS, D), jnp.float32)
    xcur = x
    kcur = N
    for i in range(5):
        knext = int(math.ceil(kcur * 0.5))
        aggs = _get_sc_conv()(xcur, src, dst, sel, zrows)
        xcur, seli, racc = _make_tc_layer(kcur, knext)(
            aggs, xcur, mask,
            Wrel[i].T, brel[i].reshape(1, D), Wroot[i].T,
            gamma[i].reshape(1, D), beta[i].reshape(1, D),
            pvec[i].reshape(D, 1), racc)
        sel = seli.reshape(N)
        mask = seli.astype(jnp.float32)
        kcur = knext
    return _tc_head(racc, W1.T, b1.reshape(1, 128), W2.T, b2.reshape(1, 64),
                    W3.T, b3.reshape(1, 10))
